# baseline (speedup 1.0000x reference)
import jax
import jax.numpy as jnp
from jax.experimental import pallas as pl
from jax.experimental.pallas import tpu as pltpu

N = 4096
BM = 512


def _rowsum_body(p_ref, a_ref, o_ref):
    o_ref[...] = jnp.sum(a_ref[...], axis=1, keepdims=True) * p_ref[0, 0]


def kernel(p, A):
    p2 = p.reshape(1, 1)
    y = pl.pallas_call(
        _rowsum_body,
        grid=(N // BM,),
        in_specs=[
            pl.BlockSpec((1, 1), lambda i: (0, 0), memory_space=pltpu.SMEM),
            pl.BlockSpec((BM, N), lambda i: (i, 0)),
        ],
        out_specs=pl.BlockSpec((BM, 1), lambda i: (i, 0)),
        out_shape=jax.ShapeDtypeStruct((N, 1), jnp.float32),
    )(p2, A)
    return y
